# baseline (speedup 1.0000x reference)
import functools

import jax
import jax.numpy as jnp
from jax import lax
from jax.experimental import pallas as pl
from jax.experimental.pallas import tpu as pltpu
from jax.experimental.pallas import tpu_sc as plsc

DIM = 1024
ROWS = 8192
NC, NS, L = 2, 16, 16
NW = NC * NS
ROWS_PER_W = ROWS // NW
CHUNK = 32
N_CHUNKS = ROWS_PER_W // CHUNK
VECS_PER_ROW = DIM // L


def _sc_body(emb_hbm, out_hbm, buf, sem_in, sem_out, *, scale):
    wid = lax.axis_index("s") * NC + lax.axis_index("c")
    base = wid * ROWS_PER_W

    def start_in(g, slot):
        return pltpu.async_copy(
            emb_hbm.at[pl.ds(base + g * CHUNK, CHUNK)], buf.at[slot], sem_in
        )

    def start_out(g, slot):
        return pltpu.async_copy(
            buf.at[slot], out_hbm.at[pl.ds(base + g * CHUNK, CHUNK)], sem_out
        )

    def wait_in(slot):
        pltpu.make_async_copy(emb_hbm.at[pl.ds(0, CHUNK)], buf.at[slot], sem_in).wait()

    def wait_out(slot):
        pltpu.make_async_copy(buf.at[slot], out_hbm.at[pl.ds(0, CHUNK)], sem_out).wait()

    def compute(slot):
        def row_body(r, carry):
            for c in range(VECS_PER_ROW):
                v = buf[slot, r, pl.ds(c * L, L)]
                buf[slot, r, pl.ds(c * L, L)] = v * scale
            return carry

        lax.fori_loop(0, CHUNK, row_body, jnp.int32(0), unroll=False)

    start_in(0, 0)
    for g in range(N_CHUNKS):
        slot = g % 2
        wait_in(slot)
        if g >= 1:
            wait_out((g - 1) % 2)
        if g + 1 < N_CHUNKS:
            start_in(g + 1, (g + 1) % 2)
        compute(slot)
        start_out(g, slot)
    wait_out((N_CHUNKS - 1) % 2)


@jax.jit
def sc_scaled_copy(emb):
    scale = DIM ** (-0.5)
    mesh = plsc.VectorSubcoreMesh(
        core_axis_name="c", subcore_axis_name="s", num_cores=NC, num_subcores=NS
    )
    return pl.kernel(
        functools.partial(_sc_body, scale=scale),
        out_type=jax.ShapeDtypeStruct((ROWS, DIM), jnp.float32),
        mesh=mesh,
        scratch_types=[
            pltpu.VMEM((2, CHUNK, DIM), jnp.float32),
            pltpu.SemaphoreType.DMA,
            pltpu.SemaphoreType.DMA,
        ],
    )(emb)


def kernel(x, emb):
    del x
    return sc_scaled_copy(emb)


# speedup vs baseline: 1.0430x; 1.0430x over previous
import functools

import jax
import jax.numpy as jnp
from jax import lax
from jax.experimental import pallas as pl
from jax.experimental.pallas import tpu as pltpu
from jax.experimental.pallas import tpu_sc as plsc

DIM = 1024
ROWS = 8192
NC, NS, L = 2, 16, 16
NW = NC * NS
ROWS_PER_W = ROWS // NW
CHUNK = 32
NBUF = 3
N_CHUNKS = ROWS_PER_W // CHUNK
VECS_PER_ROW = DIM // L


def _sc_body(emb_hbm, out_hbm, buf, sems_in, sems_out, *, scale):
    wid = lax.axis_index("s") * NC + lax.axis_index("c")
    base = wid * ROWS_PER_W

    def start_in(g, slot):
        pltpu.async_copy(
            emb_hbm.at[pl.ds(base + g * CHUNK, CHUNK)], buf.at[slot], sems_in[slot]
        )

    def start_out(g, slot):
        pltpu.async_copy(
            buf.at[slot], out_hbm.at[pl.ds(base + g * CHUNK, CHUNK)], sems_out[slot]
        )

    def wait_in(slot):
        pltpu.make_async_copy(
            emb_hbm.at[pl.ds(0, CHUNK)], buf.at[slot], sems_in[slot]
        ).wait()

    def wait_out(slot):
        pltpu.make_async_copy(
            buf.at[slot], out_hbm.at[pl.ds(0, CHUNK)], sems_out[slot]
        ).wait()

    def compute(slot):
        def row_body(r, carry):
            for c in range(VECS_PER_ROW):
                v = buf[slot, r, pl.ds(c * L, L)]
                buf[slot, r, pl.ds(c * L, L)] = v * scale
            return carry

        lax.fori_loop(0, CHUNK, row_body, jnp.int32(0))

    start_in(0, 0)
    start_in(1, 1)
    for g in range(N_CHUNKS):
        slot = g % NBUF
        wait_in(slot)
        compute(slot)
        start_out(g, slot)
        nxt = g + 2
        if nxt < N_CHUNKS:
            nslot = nxt % NBUF
            if nxt >= NBUF:
                wait_out(nslot)
            start_in(nxt, nslot)
    for g in range(max(N_CHUNKS - NBUF + 1, 0), N_CHUNKS):
        wait_out(g % NBUF)


@jax.jit
def _sc_scaled_copy(emb):
    scale = DIM ** (-0.5)
    mesh = plsc.VectorSubcoreMesh(
        core_axis_name="c", subcore_axis_name="s", num_cores=NC, num_subcores=NS
    )

    def body(emb_hbm, out_hbm, buf, si0, si1, si2, so0, so1, so2):
        _sc_body(
            emb_hbm,
            out_hbm,
            buf,
            [si0, si1, si2],
            [so0, so1, so2],
            scale=scale,
        )

    return pl.kernel(
        body,
        out_type=jax.ShapeDtypeStruct((ROWS, DIM), jnp.float32),
        mesh=mesh,
        scratch_types=[
            pltpu.VMEM((NBUF, CHUNK, DIM), jnp.float32),
            pltpu.SemaphoreType.DMA,
            pltpu.SemaphoreType.DMA,
            pltpu.SemaphoreType.DMA,
            pltpu.SemaphoreType.DMA,
            pltpu.SemaphoreType.DMA,
            pltpu.SemaphoreType.DMA,
        ],
    )(emb)


def kernel(x, emb):
    del x
    return _sc_scaled_copy(emb)
